# baseline (speedup 1.0000x reference)
import functools

import jax
import jax.numpy as jnp
import numpy as np
from jax.experimental import pallas as pl
from jax.experimental.pallas import tpu as pltpu

_STRIDES = [8, 16, 32]
_IMAGE_SIZE = 640
_FS = [_IMAGE_SIZE // s for s in _STRIDES]
_ANCHORS = [
    np.array([[12., 16.], [19., 36.], [40., 28.]], dtype=np.float32) / 8.0,
    np.array([[36., 75.], [76., 55.], [72., 146.]], dtype=np.float32) / 16.0,
    np.array([[142., 110.], [192., 243.], [459., 401.]], dtype=np.float32) / 32.0,
]
_K = 90
_HALF_MAX = 65504.0
_B, _N, _A = 128, 150, 3
_R = _A * _N


def _copy_bits(xm, ym, fs):
    b1 = (xm % 1.0 < 0.5) & (xm > 1.0)
    b2 = (ym % 1.0 < 0.5) & (ym > 1.0)
    invx = jnp.where(xm != 0.0, fs - xm, 0.0)
    invy = jnp.where(ym != 0.0, fs - ym, 0.0)
    b3 = (invx % 1.0 < 0.5) & (invx > 1.0)
    b4 = (invy % 1.0 < 0.5) & (invy > 1.0)
    return b1, b2, b3, b4


def _anchor_mask(w0, h0, aw, ah, a):
    rw = w0 / float(aw[a])
    rh = h0 / float(ah[a])
    worse = jnp.maximum(jnp.maximum(rw, 1.0 / rw), jnp.maximum(rh, 1.0 / rh))
    worse = jnp.where(worse != 0.0, worse, _HALF_MAX)
    return worse < 4.0


def _head_kernel(cls_ref, x_ref, y_ref, w_ref, h_ref,
                 clsT_ref, xT_ref, yT_ref, wT_ref, hT_ref,
                 out_ref, v_ref, key_ref, clm_ref, np_ref,
                 *, fs, aw, ah):
    f32 = jnp.float32
    i32 = jnp.int32
    BIG = jnp.int32(2 ** 30)

    cls2 = cls_ref[:, :]
    x2 = x_ref[:, :] * fs
    y2 = y_ref[:, :] * fs
    w2 = w_ref[:, :] * fs
    h2 = h_ref[:, :] * fs
    w0 = w2[:, 0:1]
    h0 = h2[:, 0:1]

    v_l, np_l = [], []
    for a in range(_A):
        maskT = _anchor_mask(w0, h0, aw, ah, a)
        xm = jnp.where(maskT, x2, 0.0)
        ym = jnp.where(maskT, y2, 0.0)
        b1, b2, b3, b4 = _copy_bits(xm, ym, fs)
        v_raw = ((((cls2 + x2) + y2) + w2) + h2) + float(a + 1)
        v_l.append(jnp.where(maskT, v_raw, -1.0))
        b1i = b1.astype(i32)
        b2i = b2.astype(i32)
        b3i = b3.astype(i32)
        b4i = b4.astype(i32)
        s1 = b1i
        s2 = s1 + b2i
        s3 = s2 + b3i
        s4 = s3 + b4i
        nth = []
        for k in range(1, 5):
            r = (1 * b1i * (s1 == k) + 2 * b2i * (s2 == k)
                 + 3 * b3i * (s3 == k) + 4 * b4i * (s4 == k))
            nth.append(jnp.where(s4 < k, 7, r))
        np_l.append(nth[0] + 8 * nth[1] + 64 * nth[2] + 512 * nth[3] + 4096 * 7)

    S = (_R, _B)
    ri = jax.lax.broadcasted_iota(i32, S, 0)
    bi = jax.lax.broadcasted_iota(i32, S, 1)
    ai = ri // _N
    nn = ri - ai * _N
    fidx = (bi * _N + nn) * _A + ai

    v_ref[:, :] = jnp.concatenate(v_l, axis=0)
    key_ref[:, :] = fidx
    clm_ref[:, :] = jnp.zeros(S, i32)
    np_ref[:, :] = jnp.concatenate(np_l, axis=0)

    clsT = clsT_ref[:, :]
    xT = xT_ref[:, :] * fs
    yT = yT_ref[:, :] * fs
    wT = wT_ref[:, :] * fs
    hT = hT_ref[:, :] * fs
    w0T = wT[0:1, :]
    h0T = hT[0:1, :]
    bT = jax.lax.broadcasted_iota(i32, (_B, _N), 0)
    nT = jax.lax.broadcasted_iota(i32, (_B, _N), 1)
    rmax_l, rkey_l = [], []
    for a in range(_A):
        maskTa = _anchor_mask(w0T, h0T, aw, ah, a)
        v_rawT = ((((clsT + xT) + yT) + wT) + hT) + float(a + 1)
        vTa = jnp.where(maskTa, v_rawT, -1.0)
        fxa = (bT * _N + nT) * _A + a
        rm = jnp.max(vTa, axis=0, keepdims=True)
        rk = jnp.min(jnp.where(vTa == rm, fxa, BIG), axis=0, keepdims=True)
        rmax_l.append(rm)
        rkey_l.append(rk)
    rowmax0 = jnp.concatenate(rmax_l, axis=1)
    rowkey0 = jnp.concatenate(rkey_l, axis=1)

    lanes1 = jax.lax.broadcasted_iota(i32, (1, _B), 1)
    laneR = jax.lax.broadcasted_iota(i32, (1, _R), 1)
    rows8 = jax.lax.broadcasted_iota(i32, (8, 128), 0)
    lanes8 = jax.lax.broadcasted_iota(i32, (8, 128), 1)

    def body(p, carry):
        rowmax, rowkey, out_acc = carry
        m = jnp.max(rowmax)
        valid = m > 0.0
        kmin = jnp.min(jnp.where(rowmax == m, rowkey, BIG))
        cmin = kmin // 65536
        widx = kmin - cmin * 65536
        b_s = widx // _N
        t_s = widx - (widx // _A) * _A
        bn = widx // _A
        a_s = t_s
        b_s = bn // _N
        n_s = bn - b_s * _N
        row = a_s * _N + n_s

        lm = lanes1 == b_s
        lmf = lm.astype(f32)
        g_cls = jnp.sum(cls_ref[pl.ds(n_s, 1), :] * lmf)
        g_x = jnp.sum(x_ref[pl.ds(n_s, 1), :] * lmf) * fs
        g_y = jnp.sum(y_ref[pl.ds(n_s, 1), :] * lmf) * fs
        g_w = jnp.sum(w_ref[pl.ds(n_s, 1), :] * lmf) * fs
        g_h = jnp.sum(h_ref[pl.ds(n_s, 1), :] * lmf) * fs
        g_a = a_s.astype(f32)

        offx = 0.5 * ((cmin == 1).astype(f32) - (cmin == 3).astype(f32))
        offy = 0.5 * ((cmin == 2).astype(f32) - (cmin == 4).astype(f32))
        lxi_x = jnp.where(g_x != 0.0, (g_x - offx).astype(i32), 0)
        lxi_y = jnp.where(g_y != 0.0, (g_y - offy).astype(i32), 0)
        x_ind = jnp.clip(lxi_x, 0, int(fs) - 1).astype(f32)
        y_ind = jnp.clip(lxi_y, 0, int(fs) - 1).astype(f32)
        tbx = g_x - lxi_x.astype(f32)
        tby = g_y - lxi_y.astype(f32)

        vf = valid.astype(f32)
        col = (g_a * (rows8 == 0) + y_ind * (rows8 == 1) + x_ind * (rows8 == 2)
               + g_cls * (rows8 == 3) + tbx * (rows8 == 4) + tby * (rows8 == 5)
               + g_w * (rows8 == 6) + g_h * (rows8 == 7)) * vf
        out_acc = jnp.where(lanes8 == p, col, out_acc)

        crow = clm_ref[pl.ds(row, 1), :]
        c_new = jnp.sum(jnp.where(lm, crow, 0)) + 1
        nprow = np_ref[pl.ds(row, 1), :]
        npv = jnp.sum(jnp.where(lm, nprow, 0))
        nc_new = jax.lax.shift_right_logical(npv, 3 * (c_new - 1)) & 7
        exhausted = nc_new == 7
        krow = key_ref[pl.ds(row, 1), :]
        krow2 = jnp.where(lm, nc_new * 65536 + widx, krow)
        vrow = v_ref[pl.ds(row, 1), :]
        vrow2 = jnp.where(lm & exhausted, -1.0, vrow)

        @pl.when(valid)
        def _():
            clm_ref[pl.ds(row, 1), :] = crow + lm.astype(i32)
            key_ref[pl.ds(row, 1), :] = krow2
            v_ref[pl.ds(row, 1), :] = vrow2

        nm = jnp.max(vrow2)
        rkn = jnp.min(jnp.where(vrow2 == nm, krow2, BIG))
        upd = valid & (laneR == row)
        rowmax = jnp.where(upd, nm, rowmax)
        rowkey = jnp.where(upd, rkn, rowkey)
        return rowmax, rowkey, out_acc

    acc0 = jnp.zeros((8, 128), f32)
    _, _, out_acc = jax.lax.fori_loop(0, _K, body, (rowmax0, rowkey0, acc0))
    out_ref[:, :] = out_acc


def _run_head(parts, partsT, h):
    fs = float(_FS[h])
    fn = functools.partial(_head_kernel, fs=fs,
                           aw=_ANCHORS[h][:, 0], ah=_ANCHORS[h][:, 1])
    out = pl.pallas_call(
        fn,
        out_shape=jax.ShapeDtypeStruct((8, 128), jnp.float32),
        scratch_shapes=[
            pltpu.VMEM((_R, _B), jnp.float32),
            pltpu.VMEM((_R, _B), jnp.int32),
            pltpu.VMEM((_R, _B), jnp.int32),
            pltpu.VMEM((_R, _B), jnp.int32),
        ],
    )(*(parts + partsT))
    anchor = out[0, :_K].astype(jnp.int32)
    y_ind = out[1, :_K].astype(jnp.int32)
    x_ind = out[2, :_K].astype(jnp.int32)
    t_boxes = out[3:8, :_K].T
    return anchor, y_ind, x_ind, t_boxes


@jax.jit
def kernel(real_labels):
    lt = real_labels.transpose(2, 1, 0)
    parts = tuple(lt[i] for i in range(5))
    ltT = real_labels.transpose(2, 0, 1)
    partsT = tuple(ltT[i] for i in range(5))
    out = ()
    for h in range(3):
        out = out + _run_head(parts, partsT, h)
    return out


# speedup vs baseline: 1.1209x; 1.1209x over previous
import jax
import jax.numpy as jnp
import numpy as np
from jax.experimental import pallas as pl
from jax.experimental.pallas import tpu as pltpu

_STRIDES = [8, 16, 32]
_IMAGE_SIZE = 640
_FS = [float(_IMAGE_SIZE // s) for s in _STRIDES]
_ANCHORS = [
    np.array([[12., 16.], [19., 36.], [40., 28.]], dtype=np.float32) / 8.0,
    np.array([[36., 75.], [76., 55.], [72., 146.]], dtype=np.float32) / 16.0,
    np.array([[142., 110.], [192., 243.], [459., 401.]], dtype=np.float32) / 32.0,
]
_K = 90
_HALF_MAX = 65504.0
_B, _N, _A, _H = 128, 150, 3, 3
_R = _A * _N


def _copy_bits(xm, ym, fs):
    b1 = (xm % 1.0 < 0.5) & (xm > 1.0)
    b2 = (ym % 1.0 < 0.5) & (ym > 1.0)
    invx = jnp.where(xm != 0.0, fs - xm, 0.0)
    invy = jnp.where(ym != 0.0, fs - ym, 0.0)
    b3 = (invx % 1.0 < 0.5) & (invx > 1.0)
    b4 = (invy % 1.0 < 0.5) & (invy > 1.0)
    return b1, b2, b3, b4


def _anchor_mask(w0, h0, h, a):
    rw = w0 / float(_ANCHORS[h][a, 0])
    rh = h0 / float(_ANCHORS[h][a, 1])
    worse = jnp.maximum(jnp.maximum(rw, 1.0 / rw), jnp.maximum(rh, 1.0 / rh))
    worse = jnp.where(worse != 0.0, worse, _HALF_MAX)
    return worse < 4.0


def _kernel(cls_ref, x_ref, y_ref, w_ref, h_ref,
            clsT_ref, xT_ref, yT_ref, wT_ref, hT_ref,
            o0_ref, o1_ref, o2_ref, v_ref, key_ref, clm_ref, np_ref):
    f32 = jnp.float32
    i32 = jnp.int32
    BIG = jnp.int32(2 ** 30)
    out_refs = (o0_ref, o1_ref, o2_ref)

    cls2 = cls_ref[:, :]
    clsT = clsT_ref[:, :]
    S = (_R, _B)
    ri = jax.lax.broadcasted_iota(i32, S, 0)
    bi = jax.lax.broadcasted_iota(i32, S, 1)
    ai = ri // _N
    nn = ri - ai * _N
    fidx = (bi * _N + nn) * _A + ai
    bT = jax.lax.broadcasted_iota(i32, (_B, _N), 0)
    nT = jax.lax.broadcasted_iota(i32, (_B, _N), 1)

    rowmax0, rowkey0 = [], []
    for h in range(_H):
        fs = _FS[h]
        x2 = x_ref[:, :] * fs
        y2 = y_ref[:, :] * fs
        w2 = w_ref[:, :] * fs
        h2 = h_ref[:, :] * fs
        w0 = w2[:, 0:1]
        h0 = h2[:, 0:1]
        v_l, np_l = [], []
        for a in range(_A):
            maskT = _anchor_mask(w0, h0, h, a)
            xm = jnp.where(maskT, x2, 0.0)
            ym = jnp.where(maskT, y2, 0.0)
            b1, b2, b3, b4 = _copy_bits(xm, ym, fs)
            v_raw = ((((cls2 + x2) + y2) + w2) + h2) + float(a + 1)
            v_l.append(jnp.where(maskT, v_raw, -1.0))
            b1i = b1.astype(i32)
            b2i = b2.astype(i32)
            b3i = b3.astype(i32)
            b4i = b4.astype(i32)
            s1 = b1i
            s2 = s1 + b2i
            s3 = s2 + b3i
            s4 = s3 + b4i
            nth = []
            for k in range(1, 5):
                r = (1 * b1i * (s1 == k) + 2 * b2i * (s2 == k)
                     + 3 * b3i * (s3 == k) + 4 * b4i * (s4 == k))
                nth.append(jnp.where(s4 < k, 7, r))
            np_l.append(nth[0] + 8 * nth[1] + 64 * nth[2]
                        + 512 * nth[3] + 4096 * 7)
        v_ref[h] = jnp.concatenate(v_l, axis=0)
        key_ref[h] = fidx
        clm_ref[h] = jnp.zeros(S, i32)
        np_ref[h] = jnp.concatenate(np_l, axis=0)

        xT = xT_ref[:, :] * fs
        yT = yT_ref[:, :] * fs
        wT = wT_ref[:, :] * fs
        hT = hT_ref[:, :] * fs
        w0T = wT[0:1, :]
        h0T = hT[0:1, :]
        rmax_l, rkey_l = [], []
        for a in range(_A):
            maskTa = _anchor_mask(w0T, h0T, h, a)
            v_rawT = ((((clsT + xT) + yT) + wT) + hT) + float(a + 1)
            vTa = jnp.where(maskTa, v_rawT, -1.0)
            fxa = (bT * _N + nT) * _A + a
            rm = jnp.max(vTa, axis=0, keepdims=True)
            rk = jnp.min(jnp.where(vTa == rm, fxa, BIG), axis=0, keepdims=True)
            rmax_l.append(rm)
            rkey_l.append(rk)
        rowmax0.append(jnp.concatenate(rmax_l, axis=1))
        rowkey0.append(jnp.concatenate(rkey_l, axis=1))

    lanes1 = jax.lax.broadcasted_iota(i32, (1, _B), 1)
    laneR = jax.lax.broadcasted_iota(i32, (1, _R), 1)
    rows8 = jax.lax.broadcasted_iota(i32, (8, 128), 0)
    lanes8 = jax.lax.broadcasted_iota(i32, (8, 128), 1)

    def body(p, carry):
        rowmax, rowkey, out_acc = carry
        new_rm, new_rk, new_oa = [], [], []
        for h in range(_H):
            fs = _FS[h]
            m = jnp.max(rowmax[h])
            valid = m > 0.0
            kmin = jnp.min(jnp.where(rowmax[h] == m, rowkey[h], BIG))
            cmin = kmin // 65536
            widx = kmin - cmin * 65536
            a_s = widx - (widx // _A) * _A
            bn = widx // _A
            b_s = bn // _N
            n_s = bn - b_s * _N
            row = a_s * _N + n_s

            lm = lanes1 == b_s
            lmf = lm.astype(f32)
            g_cls = jnp.sum(cls_ref[pl.ds(n_s, 1), :] * lmf)
            g_x = jnp.sum(x_ref[pl.ds(n_s, 1), :] * lmf) * fs
            g_y = jnp.sum(y_ref[pl.ds(n_s, 1), :] * lmf) * fs
            g_w = jnp.sum(w_ref[pl.ds(n_s, 1), :] * lmf) * fs
            g_h = jnp.sum(h_ref[pl.ds(n_s, 1), :] * lmf) * fs
            g_a = a_s.astype(f32)

            offx = 0.5 * ((cmin == 1).astype(f32) - (cmin == 3).astype(f32))
            offy = 0.5 * ((cmin == 2).astype(f32) - (cmin == 4).astype(f32))
            lxi_x = jnp.where(g_x != 0.0, (g_x - offx).astype(i32), 0)
            lxi_y = jnp.where(g_y != 0.0, (g_y - offy).astype(i32), 0)
            x_ind = jnp.clip(lxi_x, 0, int(fs) - 1).astype(f32)
            y_ind = jnp.clip(lxi_y, 0, int(fs) - 1).astype(f32)
            tbx = g_x - lxi_x.astype(f32)
            tby = g_y - lxi_y.astype(f32)

            vf = valid.astype(f32)
            col = (g_a * (rows8 == 0) + y_ind * (rows8 == 1)
                   + x_ind * (rows8 == 2) + g_cls * (rows8 == 3)
                   + tbx * (rows8 == 4) + tby * (rows8 == 5)
                   + g_w * (rows8 == 6) + g_h * (rows8 == 7)) * vf
            new_oa.append(jnp.where(lanes8 == p, col, out_acc[h]))

            crow = clm_ref[h, pl.ds(row, 1), :]
            c_new = jnp.sum(jnp.where(lm, crow, 0)) + 1
            nprow = np_ref[h, pl.ds(row, 1), :]
            npv = jnp.sum(jnp.where(lm, nprow, 0))
            nc_new = jax.lax.shift_right_logical(npv, 3 * (c_new - 1)) & 7
            exhausted = nc_new == 7
            krow = key_ref[h, pl.ds(row, 1), :]
            krow2 = jnp.where(lm, nc_new * 65536 + widx, krow)
            vrow = v_ref[h, pl.ds(row, 1), :]
            vrow2 = jnp.where(lm & exhausted, -1.0, vrow)

            @pl.when(valid)
            def _(h=h, row=row, crow=crow, lm=lm, krow2=krow2, vrow2=vrow2):
                clm_ref[h, pl.ds(row, 1), :] = crow + lm.astype(i32)
                key_ref[h, pl.ds(row, 1), :] = krow2
                v_ref[h, pl.ds(row, 1), :] = vrow2

            nm = jnp.max(vrow2)
            rkn = jnp.min(jnp.where(vrow2 == nm, krow2, BIG))
            upd = valid & (laneR == row)
            new_rm.append(jnp.where(upd, nm, rowmax[h]))
            new_rk.append(jnp.where(upd, rkn, rowkey[h]))
        return tuple(new_rm), tuple(new_rk), tuple(new_oa)

    acc0 = tuple(jnp.zeros((8, 128), f32) for _ in range(_H))
    _, _, out_acc = jax.lax.fori_loop(
        0, _K, body, (tuple(rowmax0), tuple(rowkey0), acc0))
    for h in range(_H):
        out_refs[h][:, :] = out_acc[h]


@jax.jit
def kernel(real_labels):
    lt = real_labels.transpose(2, 1, 0)
    parts = tuple(lt[i] for i in range(5))
    ltT = real_labels.transpose(2, 0, 1)
    partsT = tuple(ltT[i] for i in range(5))
    outs = pl.pallas_call(
        _kernel,
        out_shape=[jax.ShapeDtypeStruct((8, 128), jnp.float32)] * _H,
        scratch_shapes=[
            pltpu.VMEM((_H, _R, _B), jnp.float32),
            pltpu.VMEM((_H, _R, _B), jnp.int32),
            pltpu.VMEM((_H, _R, _B), jnp.int32),
            pltpu.VMEM((_H, _R, _B), jnp.int32),
        ],
    )(*(parts + partsT))
    out = ()
    for h in range(_H):
        o = outs[h]
        anchor = o[0, :_K].astype(jnp.int32)
        y_ind = o[1, :_K].astype(jnp.int32)
        x_ind = o[2, :_K].astype(jnp.int32)
        t_boxes = o[3:8, :_K].T
        out = out + (anchor, y_ind, x_ind, t_boxes)
    return out
